# baseline (speedup 1.0000x reference)
import functools

import jax
import jax.numpy as jnp
from jax import lax
from jax.experimental import pallas as pl
from jax.experimental.pallas import tpu as pltpu
from jax.experimental.pallas import tpu_sc as plsc

B = 4096
D = 64
K = 70
KP1 = 71
ROW_BLK = 256
S_BLK = 128
T_PAIRS_TSA = 4
T_PAIRS_COV = 8
BIG = 3e38
BIGI = 1 << 30


def _stats_body(latent_ref, outs_ref, tgts_ref,
                trace_ref, frob_ref, lmax_ref, recon_ref):
    lat = latent_ref[...]
    zc = lat - jnp.mean(lat, axis=0, keepdims=True)
    cov = lax.dot_general(zc, zc, (((0,), (0,)), ((), ())),
                          preferred_element_type=jnp.float32)
    trace_ref[...] = jnp.sum(zc * zc).reshape(1, 1)
    frob_ref[...] = jnp.sum(cov * cov).reshape(1, 1)

    def pstep(_, u):
        w = jnp.sum(cov * u, axis=1, keepdims=True)
        w = w * lax.rsqrt(jnp.sum(w * w) + 1e-30)
        u2 = jnp.sum(cov * w, axis=0, keepdims=True)
        u2 = u2 * lax.rsqrt(jnp.sum(u2 * u2) + 1e-30)
        return u2
    u = lax.fori_loop(0, T_PAIRS_COV, pstep, jnp.ones((1, D), jnp.float32))
    w = jnp.sum(cov * u, axis=1, keepdims=True)
    lmax_ref[...] = jnp.sqrt(jnp.sum(w * w)).reshape(1, 1)

    diff = outs_ref[...] - tgts_ref[...]
    recon_ref[...] = (jnp.sum(diff * diff) * (1.0 / (B * D))).reshape(1, 1)


_scalar_out = jax.ShapeDtypeStruct((1, 1), jnp.float32)


def _stats_call(latent, outputs, targets):
    return pl.pallas_call(
        _stats_body,
        out_shape=[_scalar_out] * 4,
    )(latent, outputs, targets)


def _knn_body(rawblk_ref, rawT_ref, idx_ref):
    rb = rawblk_ref[...]
    rT = rawT_ref[...]
    dot = lax.dot_general(rb, rT, (((1,), (0,)), ((), ())),
                          preferred_element_type=jnp.float32)
    sq_all = jnp.sum(rT * rT, axis=0, keepdims=True)
    sq_blk = jnp.sum(rb * rb, axis=1, keepdims=True)
    d2 = jnp.maximum(sq_blk + sq_all - 2.0 * dot, 0.0)
    col = lax.broadcasted_iota(jnp.int32, (ROW_BLK, B), 1)
    lane = lax.broadcasted_iota(jnp.int32, (ROW_BLK, KP1 + 1), 1)
    pk = lax.bitcast_convert_type(
        (lax.bitcast_convert_type(d2, jnp.int32) & ~0xFFF) | col, jnp.float32)

    def step(k, carry):
        pk, acc, mn = carry
        amin = lax.bitcast_convert_type(mn, jnp.int32) & 0xFFF
        acc = jnp.where(lane == k, amin, acc)
        pk = jnp.where(pk == mn, BIG, pk)
        return pk, acc, jnp.min(pk, axis=1, keepdims=True)

    acc0 = jnp.zeros((ROW_BLK, KP1 + 1), jnp.int32)
    mn0 = jnp.min(pk, axis=1, keepdims=True)
    _, acc, _ = lax.fori_loop(0, KP1, step, (pk, acc0, mn0))
    idx_ref[...] = acc


def _knn_call(raw, rawT):
    return pl.pallas_call(
        _knn_body,
        grid=(B // ROW_BLK,),
        in_specs=[
            pl.BlockSpec((ROW_BLK, D), lambda i: (i, 0)),
            pl.BlockSpec((D, B), lambda i: (0, 0)),
        ],
        out_specs=pl.BlockSpec((ROW_BLK, KP1 + 1), lambda i: (i, 0)),
        out_shape=jax.ShapeDtypeStruct((B, KP1 + 1), jnp.int32),
    )(raw, rawT)


_NC, _NS = 2, 16
_NW = _NC * _NS
_NTOT = B * K
_PER_W = _NTOT // _NW
_CH = 128
_NCH = _PER_W // _CH


def _gather_sc_body(table_hbm, idx_hbm, yout_hbm, idx_v, y_v, sem):
    wid = lax.axis_index("s") * _NC + lax.axis_index("c")

    def chunk(c, carry):
        base = wid * _PER_W + c * _CH
        pltpu.sync_copy(idx_hbm.at[pl.ds(base, _CH)], idx_v)
        pltpu.async_copy(table_hbm.at[idx_v], y_v, sem).wait()
        pltpu.sync_copy(y_v, yout_hbm.at[pl.ds(base, _CH)])
        return carry

    lax.fori_loop(0, _NCH, chunk, 0)


@functools.cache
def _gather_sc():
    return pl.kernel(
        _gather_sc_body,
        out_type=jax.ShapeDtypeStruct((_NTOT, 2 * D), jnp.float32),
        mesh=plsc.VectorSubcoreMesh(core_axis_name="c", subcore_axis_name="s"),
        scratch_types=[pltpu.VMEM((_CH,), jnp.int32),
                       pltpu.VMEM((_CH, 2 * D), jnp.float32),
                       pltpu.SemaphoreType.DMA],
    )


def _tsa_body(yg_ref, out_ref):
    Y = yg_ref[...]
    ones = jnp.ones((S_BLK, K, 1), jnp.float32)

    def gramb(Yp):
        G = lax.dot_general(Yp, Yp, (((1,), (1,)), ((0,), (0,))),
                            preferred_element_type=jnp.float32)
        mcol = lax.dot_general(Yp, ones, (((1,), (1,)), ((0,), (0,))),
                               preferred_element_type=jnp.float32)
        srow = lax.dot_general(ones, Yp, (((1,), (1,)), ((0,), (0,))),
                               preferred_element_type=jnp.float32)
        return G - (mcol * srow) * (1.0 / K)

    Gz = gramb(Y[:, :, :D])
    Gx = gramb(Y[:, :, D:])

    def piter(G):
        def pstep(_, u):
            w = jnp.sum(G * u, axis=2, keepdims=True)
            n = jnp.sum(jnp.sum(w * w, axis=1, keepdims=True),
                        axis=2, keepdims=True)
            w = w * lax.rsqrt(n + 1e-30)
            u2 = jnp.sum(G * w, axis=1, keepdims=True)
            n2 = jnp.sum(jnp.sum(u2 * u2, axis=2, keepdims=True),
                         axis=1, keepdims=True)
            return u2 * lax.rsqrt(n2 + 1e-30)
        u0 = jnp.ones((S_BLK, 1, D), jnp.float32)
        return lax.fori_loop(0, T_PAIRS_TSA, pstep, u0)

    uz = piter(Gz)
    ux = piter(Gx)
    dot = jnp.sum(uz * ux, axis=2)
    part = jnp.sum(2.0 - 2.0 * dot * dot)

    @pl.when(pl.program_id(0) == 0)
    def _():
        out_ref[...] = jnp.zeros((1, 1), jnp.float32)
    out_ref[...] = out_ref[...] + part.reshape(1, 1)


def _tsa_call(yg):
    return pl.pallas_call(
        _tsa_body,
        grid=(B // S_BLK,),
        in_specs=[
            pl.BlockSpec((S_BLK, K, 2 * D), lambda i: (i, 0, 0)),
        ],
        out_specs=pl.BlockSpec((1, 1), lambda i: (0, 0)),
        out_shape=jax.ShapeDtypeStruct((1, 1), jnp.float32),
    )(yg)


def kernel(outputs, targets, latent, raw):
    trace, frob2, lmax, recon = _stats_call(latent, outputs, targets)
    idx = _knn_call(raw, raw.T)
    nbr = idx[:, 1:KP1].reshape(-1)
    table = jnp.concatenate([latent, raw], axis=1)
    yg = _gather_sc()(table, nbr)
    tsa_sum = _tsa_call(yg.reshape(B, K, 2 * D))[0, 0]

    trace = trace[0, 0]
    pr_val = trace * trace / frob2[0, 0]
    aniso_val = lmax[0, 0] / trace
    return (recon[0, 0] + 0.01 * pr_val + 0.01 * (1.0 - aniso_val)
            + 0.1 * (tsa_sum * (1.0 / B)))


# speedup vs baseline: 1.0188x; 1.0188x over previous
import functools

import jax
import jax.numpy as jnp
from jax import lax
from jax.experimental import pallas as pl
from jax.experimental.pallas import tpu as pltpu
from jax.experimental.pallas import tpu_sc as plsc

B = 4096
D = 64
K = 70
KP1 = 71
ROW_BLK = 128
S_BLK = 128
T_PAIRS_TSA = 4
T_PAIRS_COV = 8
BIG = 3e38
BIGI = 1 << 30


def _stats_body(latent_ref, outs_ref, tgts_ref,
                trace_ref, frob_ref, lmax_ref, recon_ref):
    lat = latent_ref[...]
    zc = lat - jnp.mean(lat, axis=0, keepdims=True)
    cov = lax.dot_general(zc, zc, (((0,), (0,)), ((), ())),
                          preferred_element_type=jnp.float32)
    trace_ref[...] = jnp.sum(zc * zc).reshape(1, 1)
    frob_ref[...] = jnp.sum(cov * cov).reshape(1, 1)

    def pstep(_, u):
        w = jnp.sum(cov * u, axis=1, keepdims=True)
        w = w * lax.rsqrt(jnp.sum(w * w) + 1e-30)
        u2 = jnp.sum(cov * w, axis=0, keepdims=True)
        u2 = u2 * lax.rsqrt(jnp.sum(u2 * u2) + 1e-30)
        return u2
    u = lax.fori_loop(0, T_PAIRS_COV, pstep, jnp.ones((1, D), jnp.float32))
    w = jnp.sum(cov * u, axis=1, keepdims=True)
    lmax_ref[...] = jnp.sqrt(jnp.sum(w * w)).reshape(1, 1)

    diff = outs_ref[...] - tgts_ref[...]
    recon_ref[...] = (jnp.sum(diff * diff) * (1.0 / (B * D))).reshape(1, 1)


_scalar_out = jax.ShapeDtypeStruct((1, 1), jnp.float32)


def _stats_call(latent, outputs, targets):
    return pl.pallas_call(
        _stats_body,
        out_shape=[_scalar_out] * 4,
    )(latent, outputs, targets)


def _knn_body(a_ref, bt_ref, idx_ref):
    dotv = lax.dot_general(a_ref[...], bt_ref[...], (((1,), (0,)), ((), ())),
                           preferred_element_type=jnp.float32)
    col = lax.broadcasted_iota(jnp.int32, (ROW_BLK, B), 1)
    lane = lax.broadcasted_iota(jnp.int32, (ROW_BLK, KP1 + 1), 1)
    pk = lax.bitcast_convert_type(
        (lax.bitcast_convert_type(dotv, jnp.int32) & ~0xFFF) | col,
        jnp.float32)

    def step(k, carry):
        pk, acc, mn = carry
        amin = lax.bitcast_convert_type(mn, jnp.int32) & 0xFFF
        acc = jnp.where(lane == k, amin, acc)
        pk = jnp.where(pk == mn, BIG, pk)
        return pk, acc, jnp.min(pk, axis=1, keepdims=True)

    acc0 = jnp.zeros((ROW_BLK, KP1 + 1), jnp.int32)
    mn0 = jnp.min(pk, axis=1, keepdims=True)
    _, acc, _ = lax.fori_loop(0, KP1, step, (pk, acc0, mn0))
    idx_ref[...] = acc


def _knn_call(a2, bt2):
    return pl.pallas_call(
        _knn_body,
        grid=(B // ROW_BLK,),
        in_specs=[
            pl.BlockSpec((ROW_BLK, D + 1), lambda i: (i, 0)),
            pl.BlockSpec((D + 1, B), lambda i: (0, 0)),
        ],
        out_specs=pl.BlockSpec((ROW_BLK, KP1 + 1), lambda i: (i, 0)),
        out_shape=jax.ShapeDtypeStruct((B, KP1 + 1), jnp.int32),
    )(a2, bt2)


_NC, _NS = 2, 16
_NW = _NC * _NS
_NTOT = B * K
_PER_W = _NTOT // _NW
_CH = 128
_NCH = _PER_W // _CH


def _gather_sc_body(table_hbm, idx_hbm, yout_hbm, idx_v, y_v, sem):
    wid = lax.axis_index("s") * _NC + lax.axis_index("c")

    def chunk(c, carry):
        base = wid * _PER_W + c * _CH
        pltpu.sync_copy(idx_hbm.at[pl.ds(base, _CH)], idx_v)
        pltpu.async_copy(table_hbm.at[idx_v], y_v, sem).wait()
        pltpu.sync_copy(y_v, yout_hbm.at[pl.ds(base, _CH)])
        return carry

    lax.fori_loop(0, _NCH, chunk, 0)


@functools.cache
def _gather_sc():
    return pl.kernel(
        _gather_sc_body,
        out_type=jax.ShapeDtypeStruct((_NTOT, 2 * D), jnp.float32),
        mesh=plsc.VectorSubcoreMesh(core_axis_name="c", subcore_axis_name="s"),
        scratch_types=[pltpu.VMEM((_CH,), jnp.int32),
                       pltpu.VMEM((_CH, 2 * D), jnp.float32),
                       pltpu.SemaphoreType.DMA],
    )


def _tsa_body(yg_ref, out_ref):
    Y = yg_ref[...]
    ones = jnp.ones((S_BLK, K, 1), jnp.float32)

    def gramb(Yp):
        G = lax.dot_general(Yp, Yp, (((1,), (1,)), ((0,), (0,))),
                            preferred_element_type=jnp.float32)
        mcol = lax.dot_general(Yp, ones, (((1,), (1,)), ((0,), (0,))),
                               preferred_element_type=jnp.float32)
        srow = lax.dot_general(ones, Yp, (((1,), (1,)), ((0,), (0,))),
                               preferred_element_type=jnp.float32)
        return G - (mcol * srow) * (1.0 / K)

    Gz = gramb(Y[:, :, :D])
    Gx = gramb(Y[:, :, D:])

    def piter(G):
        def pstep(_, u):
            w = jnp.sum(G * u, axis=2, keepdims=True)
            n = jnp.sum(jnp.sum(w * w, axis=1, keepdims=True),
                        axis=2, keepdims=True)
            w = w * lax.rsqrt(n + 1e-30)
            u2 = jnp.sum(G * w, axis=1, keepdims=True)
            n2 = jnp.sum(jnp.sum(u2 * u2, axis=2, keepdims=True),
                         axis=1, keepdims=True)
            return u2 * lax.rsqrt(n2 + 1e-30)
        u0 = jnp.ones((S_BLK, 1, D), jnp.float32)
        return lax.fori_loop(0, T_PAIRS_TSA, pstep, u0)

    uz = piter(Gz)
    ux = piter(Gx)
    dot = jnp.sum(uz * ux, axis=2)
    part = jnp.sum(2.0 - 2.0 * dot * dot)

    @pl.when(pl.program_id(0) == 0)
    def _():
        out_ref[...] = jnp.zeros((1, 1), jnp.float32)
    out_ref[...] = out_ref[...] + part.reshape(1, 1)


def _tsa_call(yg):
    return pl.pallas_call(
        _tsa_body,
        grid=(B // S_BLK,),
        in_specs=[
            pl.BlockSpec((S_BLK, K, 2 * D), lambda i: (i, 0, 0)),
        ],
        out_specs=pl.BlockSpec((1, 1), lambda i: (0, 0)),
        out_shape=jax.ShapeDtypeStruct((1, 1), jnp.float32),
    )(yg)


def kernel(outputs, targets, latent, raw):
    trace, frob2, lmax, recon = _stats_call(latent, outputs, targets)
    a2 = jnp.concatenate([raw * (-2.0), jnp.ones((B, 1), jnp.float32)], 1)
    bt2 = jnp.concatenate([raw.T, jnp.sum(raw * raw, 1)[None, :]], 0)
    idx = _knn_call(a2, bt2)
    nbr = idx[:, 1:KP1].reshape(-1)
    table = jnp.concatenate([latent, raw], axis=1)
    yg = _gather_sc()(table, nbr)
    tsa_sum = _tsa_call(yg.reshape(B, K, 2 * D))[0, 0]

    trace = trace[0, 0]
    pr_val = trace * trace / frob2[0, 0]
    aniso_val = lmax[0, 0] / trace
    return (recon[0, 0] + 0.01 * pr_val + 0.01 * (1.0 - aniso_val)
            + 0.1 * (tsa_sum * (1.0 / B)))
